# Initial kernel scaffold; baseline (speedup 1.0000x reference)
import jax
import jax.numpy as jnp
from jax.experimental import pallas as pl


def kernel(X, pad_vector, embedding_table):
    raise NotImplementedError("write your pallas kernel here")


# speedup vs baseline: 5.2526x; 5.2526x over previous
import functools

import jax
import jax.numpy as jnp
from jax import lax
from jax.experimental import pallas as pl
from jax.experimental.pallas import tpu as pltpu
from jax.experimental.pallas import tpu_sc as plsc

_VOCAB = 100000
_D = 64
_B = 1024 * 20 * 50

_NC = 2
_NS = 16
_NW = _NC * _NS

_PER_W = _B // _NW
_CH = 128
_N_CH = _PER_W // _CH


def _body(idx_hbm, table_hbm, out_hbm, idx_v, rows_v, gsem):
    wid = lax.axis_index("s") * _NC + lax.axis_index("c")
    base = wid * _PER_W
    pltpu.sync_copy(idx_hbm.at[wid], idx_v)

    def step(j, carry):
        pltpu.async_copy(table_hbm.at[idx_v.at[j]], rows_v, gsem).wait()
        pltpu.sync_copy(rows_v, out_hbm.at[pl.ds(base + j * _CH, _CH)])
        return carry

    lax.fori_loop(0, _N_CH, step, 0)


@functools.partial(jax.jit, static_argnums=())
def _gather(idx, table):
    mesh = plsc.VectorSubcoreMesh(
        core_axis_name="c", subcore_axis_name="s",
        num_cores=_NC, num_subcores=_NS)
    f = pl.kernel(
        _body,
        out_type=jax.ShapeDtypeStruct((_B, _D), jnp.float32),
        mesh=mesh,
        scratch_types=[
            pltpu.VMEM((_N_CH, _CH), jnp.int32),
            pltpu.VMEM((_CH, _D), jnp.float32),
            pltpu.SemaphoreType.DMA,
        ],
        compiler_params=pltpu.CompilerParams(use_tc_tiling_on_sc=False),
    )
    return f(idx, table)


def kernel(X, pad_vector, embedding_table):
    idx = X.reshape(_NW, _N_CH, _CH).astype(jnp.int32)
    out = _gather(idx, embedding_table)
    return out.reshape(X.shape + (_D,))


# speedup vs baseline: 6.2959x; 1.1986x over previous
import functools

import jax
import jax.numpy as jnp
from jax import lax
from jax.experimental import pallas as pl
from jax.experimental.pallas import tpu as pltpu
from jax.experimental.pallas import tpu_sc as plsc

_VOCAB = 100000
_D = 64
_B = 1024 * 20 * 50

_NC = 2
_NS = 16
_NW = _NC * _NS

_PER_W = _B // _NW
_CH = 128
_N_CH = _PER_W // _CH
_K = 5
_GROUP = _K * _CH
_NG = _N_CH // _K


def _body(idx_hbm, table_hbm, out_hbm,
          idx_v, rows0, rows1, g0s, g1s, w0s, w1s):
    wid = lax.axis_index("s") * _NC + lax.axis_index("c")
    base = wid * _PER_W
    pltpu.sync_copy(idx_hbm.at[wid], idx_v)

    def fire_g(g, rows, sem):
        for b in range(_K):
            pltpu.async_copy(table_hbm.at[idx_v.at[g * _K + b]],
                             rows.at[pl.ds(b * _CH, _CH)], sem)

    def drain_g(g, rows, sem):
        for b in range(_K):
            pltpu.make_async_copy(table_hbm.at[idx_v.at[g * _K + b]],
                                  rows.at[pl.ds(b * _CH, _CH)], sem).wait()

    def fire_w(g, rows, sem):
        pltpu.async_copy(rows, out_hbm.at[pl.ds(base + g * _GROUP, _GROUP)],
                         sem)

    def drain_w(g, rows, sem):
        pltpu.make_async_copy(rows,
                              out_hbm.at[pl.ds(base + g * _GROUP, _GROUP)],
                              sem).wait()

    fire_g(0, rows0, g0s)
    drain_g(0, rows0, g0s)
    fire_g(1, rows1, g1s)
    fire_w(0, rows0, w0s)

    def pair(t, carry):
        ga = 2 * t + 1
        gb = 2 * t + 2
        drain_g(ga, rows1, g1s)
        drain_w(ga - 1, rows0, w0s)
        fire_g(gb, rows0, g0s)
        fire_w(ga, rows1, w1s)
        drain_g(gb, rows0, g0s)
        drain_w(gb - 1, rows1, w1s)
        fire_g(gb + 1, rows1, g1s)
        fire_w(gb, rows0, w0s)
        return carry

    lax.fori_loop(0, _NG // 2 - 1, pair, 0)

    drain_g(_NG - 1, rows1, g1s)
    drain_w(_NG - 2, rows0, w0s)
    fire_w(_NG - 1, rows1, w1s)
    drain_w(_NG - 1, rows1, w1s)


@jax.jit
def _gather(idx, table):
    mesh = plsc.VectorSubcoreMesh(
        core_axis_name="c", subcore_axis_name="s",
        num_cores=_NC, num_subcores=_NS)
    f = pl.kernel(
        _body,
        out_type=jax.ShapeDtypeStruct((_B, _D), jnp.float32),
        mesh=mesh,
        scratch_types=[
            pltpu.VMEM((_N_CH, _CH), jnp.int32),
            pltpu.VMEM((_GROUP, _D), jnp.float32),
            pltpu.VMEM((_GROUP, _D), jnp.float32),
            pltpu.SemaphoreType.DMA,
            pltpu.SemaphoreType.DMA,
            pltpu.SemaphoreType.DMA,
            pltpu.SemaphoreType.DMA,
        ],
        compiler_params=pltpu.CompilerParams(use_tc_tiling_on_sc=False),
    )
    return f(idx, table)


def kernel(X, pad_vector, embedding_table):
    idx = X.reshape(_NW, _N_CH, _CH).astype(jnp.int32)
    out = _gather(idx, embedding_table)
    return out.reshape(X.shape + (_D,))
